# Initial kernel scaffold; baseline (speedup 1.0000x reference)
import jax
import jax.numpy as jnp
from jax.experimental import pallas as pl


def kernel(graph, capacity, ninf_mask, encoded_nodes, Wq, Wk, Wv, Wcomb, w_gate, ew1, eb1, ew2, eb2, gamma, beta):
    raise NotImplementedError("write your pallas kernel here")


# speedup vs baseline: 1.3616x; 1.3616x over previous
import jax
import jax.numpy as jnp
from jax.experimental import pallas as pl
from jax.experimental.pallas import tpu as pltpu

B, G, P = 2, 1024, 1024
EMB, HEADS, QKV = 1024, 16, 64
HQ = HEADS * QKV
E, TOPK, HID = 8, 2, 512
SQRT_EMB, CLIP = 32.0, 10.0
GT = 256
NG = G // GT
NT = 256
NTILES = (B * G) // NT
TPB = G // NT

F32 = jnp.float32
BF16 = jnp.bfloat16


def _bf(x):
    return x.astype(BF16)


def _mm(a, b):
    return jnp.dot(_bf(a), _bf(b), preferred_element_type=F32)


def _prep_body(graph_ref, enc_ref, encT_ref, wk_ref, wq_ref, wv_ref,
               kT_ref, v_ref, bq_ref):
    kf = _mm(enc_ref[0], wk_ref[...])
    kT_ref[0] = _bf(jnp.transpose(kf))
    v_ref[0] = _bf(_mm(enc_ref[0], wv_ref[...]))
    gb = jnp.broadcast_to(graph_ref[0], (128, EMB))
    bq_ref[0] = _mm(gb, wq_ref[:EMB, :])[0:1, :]


def _prep(graph, enc, encT, Wk, Wq, Wv):
    return pl.pallas_call(
        _prep_body,
        grid=(B,),
        in_specs=[
            pl.BlockSpec((1, 1, EMB), lambda b: (b, 0, 0)),
            pl.BlockSpec((1, P, EMB), lambda b: (b, 0, 0)),
            pl.BlockSpec((1, EMB, P), lambda b: (b, 0, 0)),
            pl.BlockSpec((EMB, HQ), lambda b: (0, 0)),
            pl.BlockSpec((EMB + 1, HQ), lambda b: (0, 0)),
            pl.BlockSpec((EMB, HQ), lambda b: (0, 0)),
        ],
        out_specs=[
            pl.BlockSpec((1, HQ, P), lambda b: (b, 0, 0)),
            pl.BlockSpec((1, P, HQ), lambda b: (b, 0, 0)),
            pl.BlockSpec((1, 1, HQ), lambda b: (b, 0, 0)),
        ],
        out_shape=[
            jax.ShapeDtypeStruct((B, HQ, P), BF16),
            jax.ShapeDtypeStruct((B, P, HQ), BF16),
            jax.ShapeDtypeStruct((B, 1, HQ), F32),
        ],
    )(graph, enc, encT, Wk, Wq, Wv)


def _attn_body(kT_ref, v_ref, bq_ref, wl_ref, cap_ref, mask_ref, wcomb_ref,
               wg_ref, mh_ref, gates_ref, imp_ref, acc_ref):
    b = pl.program_id(0)
    g = pl.program_id(1)
    cap = cap_ref[0]
    mask = mask_ref[0]
    capb = _bf(cap).astype(F32)
    wlb = _bf(wl_ref[...]).astype(F32)
    q = _bf(bq_ref[0] + capb * wlb)
    for h in range(HEADS):
        sl = slice(h * QKV, (h + 1) * QKV)
        score = jnp.dot(q[:, sl], kT_ref[0, sl, :],
                        preferred_element_type=F32) * 0.125 + mask
        m = jnp.max(score, axis=1, keepdims=True)
        ex = jnp.exp(score - m)
        s = jnp.sum(ex, axis=1, keepdims=True)
        acc_ref[:, sl] = jnp.dot(_bf(ex), v_ref[0, :, sl],
                                 preferred_element_type=F32) / s
    mh = _mm(acc_ref[...], wcomb_ref[...])
    mh_ref[...] = mh
    logits = _mm(mh, wg_ref[...])
    io = jax.lax.broadcasted_iota(jnp.int32, (GT, E), 1)
    m1 = jnp.max(logits, axis=1, keepdims=True)
    i1 = jnp.min(jnp.where(logits == m1, io, E), axis=1, keepdims=True)
    l2 = jnp.where(io == i1, -jnp.inf, logits)
    m2 = jnp.max(l2, axis=1, keepdims=True)
    i2 = jnp.min(jnp.where(l2 == m2, io, E), axis=1, keepdims=True)
    eg = jnp.exp(m2 - m1)
    den = 1.0 + eg
    gates = jnp.where(io == i1, 1.0 / den, 0.0) \
        + jnp.where(io == i2, eg / den, 0.0)
    gates_ref[...] = gates

    @pl.when(jnp.logical_and(b == 0, g == 0))
    def _():
        imp_ref[...] = jnp.zeros_like(imp_ref)

    imp_ref[...] += jnp.sum(gates, axis=0, keepdims=True)


def _attn(kT, v, bq, wl, cap_col, mask, Wcomb, w_gate):
    return pl.pallas_call(
        _attn_body,
        grid=(B, NG),
        in_specs=[
            pl.BlockSpec((1, HQ, P), lambda b, g: (b, 0, 0)),
            pl.BlockSpec((1, P, HQ), lambda b, g: (b, 0, 0)),
            pl.BlockSpec((1, 1, HQ), lambda b, g: (b, 0, 0)),
            pl.BlockSpec((1, HQ), lambda b, g: (0, 0)),
            pl.BlockSpec((1, GT, 1), lambda b, g: (b, g, 0)),
            pl.BlockSpec((1, GT, P), lambda b, g: (b, g, 0)),
            pl.BlockSpec((HQ, EMB), lambda b, g: (0, 0)),
            pl.BlockSpec((EMB, E), lambda b, g: (0, 0)),
        ],
        out_specs=[
            pl.BlockSpec((GT, EMB), lambda b, g: (b * NG + g, 0)),
            pl.BlockSpec((GT, E), lambda b, g: (b * NG + g, 0)),
            pl.BlockSpec((1, E), lambda b, g: (0, 0)),
        ],
        out_shape=[
            jax.ShapeDtypeStruct((B * G, EMB), F32),
            jax.ShapeDtypeStruct((B * G, E), F32),
            jax.ShapeDtypeStruct((1, E), F32),
        ],
        scratch_shapes=[pltpu.VMEM((GT, HQ), F32)],
    )(kT, v, bq, wl, cap_col, mask, Wcomb, w_gate)


def _moe_body(x_ref, gates_ref, ew1_ref, eb1_ref, ew2_ref, eb2_ref,
              added_ref, ssum_ref, acc_ref):
    n = pl.program_id(0)
    e = pl.program_id(1)
    io = jax.lax.broadcasted_iota(jnp.int32, (NT, E), 1)
    gcol = jnp.sum(jnp.where(io == e, gates_ref[...], 0.0),
                   axis=1, keepdims=True)
    x = x_ref[...]
    h = jnp.maximum(_mm(x, ew1_ref[0]) + eb1_ref[0], 0.0)
    y = _mm(h, ew2_ref[0]) + eb2_ref[0]

    @pl.when(e == 0)
    def _():
        acc_ref[...] = gcol * y

    @pl.when(e > 0)
    def _():
        acc_ref[...] += gcol * y

    @pl.when(e == E - 1)
    def _():
        added = x + acc_ref[...]
        added_ref[...] = added
        ls = jnp.sum(added, axis=0, keepdims=True)

        @pl.when(n % TPB == 0)
        def _():
            ssum_ref[0] = ls

        @pl.when(n % TPB > 0)
        def _():
            ssum_ref[0] += ls


def _moe(x, gates, ew1, eb1, ew2, eb2):
    return pl.pallas_call(
        _moe_body,
        grid=(NTILES, E),
        in_specs=[
            pl.BlockSpec((NT, EMB), lambda n, e: (n, 0)),
            pl.BlockSpec((NT, E), lambda n, e: (n, 0)),
            pl.BlockSpec((1, EMB, HID), lambda n, e: (e, 0, 0)),
            pl.BlockSpec((1, 1, HID), lambda n, e: (e, 0, 0)),
            pl.BlockSpec((1, HID, EMB), lambda n, e: (e, 0, 0)),
            pl.BlockSpec((1, 1, EMB), lambda n, e: (e, 0, 0)),
        ],
        out_specs=[
            pl.BlockSpec((NT, EMB), lambda n, e: (n, 0)),
            pl.BlockSpec((1, 1, EMB), lambda n, e: (n // TPB, 0, 0)),
        ],
        out_shape=[
            jax.ShapeDtypeStruct((B * G, EMB), F32),
            jax.ShapeDtypeStruct((B, 1, EMB), F32),
        ],
        scratch_shapes=[pltpu.VMEM((NT, EMB), F32)],
    )(x, gates, ew1, eb1[:, None, :], ew2, eb2[:, None, :])


def _stats_body(added_ref, ssum_ref, ssq_ref):
    g = pl.program_id(1)
    mu = ssum_ref[0] * (1.0 / G)
    d = added_ref[...] - mu
    lq = jnp.sum(d * d, axis=0, keepdims=True)

    @pl.when(g == 0)
    def _():
        ssq_ref[0] = lq

    @pl.when(g > 0)
    def _():
        ssq_ref[0] += lq


def _stats(added, ssum):
    return pl.pallas_call(
        _stats_body,
        grid=(B, NG),
        in_specs=[
            pl.BlockSpec((GT, EMB), lambda b, g: (b * NG + g, 0)),
            pl.BlockSpec((1, 1, EMB), lambda b, g: (b, 0, 0)),
        ],
        out_specs=pl.BlockSpec((1, 1, EMB), lambda b, g: (b, 0, 0)),
        out_shape=jax.ShapeDtypeStruct((B, 1, EMB), F32),
    )(added, ssum)


def _final_body(added_ref, ssum_ref, ssq_ref, gamma_ref, beta_ref, encT_ref,
                mask_ref, imp_ref, probs_ref, loss_ref):
    b = pl.program_id(0)
    g = pl.program_id(1)
    mu = ssum_ref[0] * (1.0 / G)
    var = ssq_ref[0] * (1.0 / G)
    rstd = jax.lax.rsqrt(var + 1e-5)
    mho = (added_ref[...] - mu) * (rstd * gamma_ref[...]) + beta_ref[...]
    sc = _mm(mho, encT_ref[0]) * (1.0 / SQRT_EMB)
    scm = CLIP * jnp.tanh(sc) + mask_ref[0]
    m = jnp.max(scm, axis=1, keepdims=True)
    ex = jnp.exp(scm - m)
    s = jnp.sum(ex, axis=1, keepdims=True)
    probs_ref[0] = ex / s

    @pl.when(jnp.logical_and(b == 0, g == 0))
    def _():
        im = imp_ref[...]
        mn = jnp.sum(im, axis=1, keepdims=True) * (1.0 / E)
        vr = jnp.sum((im - mn) ** 2, axis=1, keepdims=True) * (1.0 / E)
        loss_ref[...] = vr / (mn * mn + 1e-10)


def _final(added, ssum, ssq, gamma2, beta2, encT, mask, imp):
    return pl.pallas_call(
        _final_body,
        grid=(B, NG),
        in_specs=[
            pl.BlockSpec((GT, EMB), lambda b, g: (b * NG + g, 0)),
            pl.BlockSpec((1, 1, EMB), lambda b, g: (b, 0, 0)),
            pl.BlockSpec((1, 1, EMB), lambda b, g: (b, 0, 0)),
            pl.BlockSpec((1, EMB), lambda b, g: (0, 0)),
            pl.BlockSpec((1, EMB), lambda b, g: (0, 0)),
            pl.BlockSpec((1, EMB, P), lambda b, g: (b, 0, 0)),
            pl.BlockSpec((1, GT, P), lambda b, g: (b, g, 0)),
            pl.BlockSpec((1, E), lambda b, g: (0, 0)),
        ],
        out_specs=[
            pl.BlockSpec((1, GT, P), lambda b, g: (b, g, 0)),
            pl.BlockSpec((1, 1), lambda b, g: (0, 0)),
        ],
        out_shape=[
            jax.ShapeDtypeStruct((B, G, P), F32),
            jax.ShapeDtypeStruct((1, 1), F32),
        ],
    )(added, ssum, ssq, gamma2, beta2, encT, mask, imp)


def kernel(graph, capacity, ninf_mask, encoded_nodes, Wq, Wk, Wv, Wcomb,
           w_gate, ew1, eb1, ew2, eb2, gamma, beta):
    encT = encoded_nodes.transpose(0, 2, 1)
    kT, v, bq = _prep(graph, encoded_nodes, encT, Wk, Wq, Wv)
    wl = Wq[EMB:EMB + 1, :]
    cap_col = capacity[:, :, None]
    mh, gates, imp = _attn(kT, v, bq, wl, cap_col, ninf_mask, Wcomb, w_gate)
    added, ssum = _moe(mh, gates, ew1, eb1, ew2, eb2)
    ssq = _stats(added, ssum)
    probs, loss = _final(added, ssum, ssq, gamma[None, :], beta[None, :],
                         encT, ninf_mask, imp)
    return probs, loss[0, 0]


# speedup vs baseline: 1.7109x; 1.2565x over previous
import jax
import jax.numpy as jnp
from jax.experimental import pallas as pl
from jax.experimental.pallas import tpu as pltpu

B, G, P = 2, 1024, 1024
EMB, HEADS, QKV = 1024, 16, 64
HQ = HEADS * QKV
E, TOPK, HID = 8, 2, 512
SQRT_EMB, CLIP = 32.0, 10.0
GT = 256
NG = G // GT
NT = 256
NTILES = (B * G) // NT
TPB = G // NT

F32 = jnp.float32
BF16 = jnp.bfloat16


def _bf(x):
    return x.astype(BF16)


def _mm(a, b):
    return jnp.dot(_bf(a), _bf(b), preferred_element_type=F32)


def _prep_body(graph_ref, enc_ref, encT_ref, wk_ref, wq_ref, wv_ref,
               kT_ref, v_ref, bq_ref):
    kf = _mm(enc_ref[0], wk_ref[...])
    kT_ref[0] = _bf(jnp.transpose(kf))
    v_ref[0] = _bf(_mm(enc_ref[0], wv_ref[...]))
    gb = jnp.broadcast_to(graph_ref[0], (128, EMB))
    bq_ref[0] = _mm(gb, wq_ref[:EMB, :])[0:1, :]


def _prep(graph, enc, encT, Wk, Wq, Wv):
    return pl.pallas_call(
        _prep_body,
        grid=(B,),
        in_specs=[
            pl.BlockSpec((1, 1, EMB), lambda b: (b, 0, 0)),
            pl.BlockSpec((1, P, EMB), lambda b: (b, 0, 0)),
            pl.BlockSpec((1, EMB, P), lambda b: (b, 0, 0)),
            pl.BlockSpec((EMB, HQ), lambda b: (0, 0)),
            pl.BlockSpec((EMB + 1, HQ), lambda b: (0, 0)),
            pl.BlockSpec((EMB, HQ), lambda b: (0, 0)),
        ],
        out_specs=[
            pl.BlockSpec((1, HQ, P), lambda b: (b, 0, 0)),
            pl.BlockSpec((1, P, HQ), lambda b: (b, 0, 0)),
            pl.BlockSpec((1, 1, HQ), lambda b: (b, 0, 0)),
        ],
        out_shape=[
            jax.ShapeDtypeStruct((B, HQ, P), BF16),
            jax.ShapeDtypeStruct((B, P, HQ), BF16),
            jax.ShapeDtypeStruct((B, 1, HQ), F32),
        ],
    )(graph, enc, encT, Wk, Wq, Wv)


def _attn_body(kT_ref, v_ref, bq_ref, wl_ref, cap_ref, mask_ref, wcomb_ref,
               wg_ref, mh_ref, gates_ref, imp_ref, acc_ref):
    b = pl.program_id(0)
    g = pl.program_id(1)
    cap = cap_ref[0]
    mask = mask_ref[0]
    capb = _bf(cap).astype(F32)
    wlb = _bf(wl_ref[...]).astype(F32)
    q = _bf(bq_ref[0] + capb * wlb)
    for h in range(HEADS):
        sl = slice(h * QKV, (h + 1) * QKV)
        score = jnp.dot(q[:, sl], kT_ref[0, sl, :],
                        preferred_element_type=F32) * 0.125 + mask
        m = jnp.max(score, axis=1, keepdims=True)
        ex = jnp.exp(score - m)
        s = jnp.sum(ex, axis=1, keepdims=True)
        acc_ref[:, sl] = jnp.dot(_bf(ex), v_ref[0, :, sl],
                                 preferred_element_type=F32) / s
    mh = _mm(acc_ref[...], wcomb_ref[...])
    mh_ref[...] = mh
    logits = _mm(mh, wg_ref[...])
    io = jax.lax.broadcasted_iota(jnp.int32, (GT, E), 1)
    m1 = jnp.max(logits, axis=1, keepdims=True)
    i1 = jnp.min(jnp.where(logits == m1, io, E), axis=1, keepdims=True)
    l2 = jnp.where(io == i1, -jnp.inf, logits)
    m2 = jnp.max(l2, axis=1, keepdims=True)
    i2 = jnp.min(jnp.where(l2 == m2, io, E), axis=1, keepdims=True)
    eg = jnp.exp(m2 - m1)
    den = 1.0 + eg
    gates = jnp.where(io == i1, 1.0 / den, 0.0) \
        + jnp.where(io == i2, eg / den, 0.0)
    gates_ref[...] = gates

    @pl.when(jnp.logical_and(b == 0, g == 0))
    def _():
        imp_ref[...] = jnp.zeros_like(imp_ref)

    imp_ref[...] += jnp.sum(gates, axis=0, keepdims=True)


def _attn(kT, v, bq, wl, cap_col, mask, Wcomb, w_gate):
    return pl.pallas_call(
        _attn_body,
        grid=(B, NG),
        in_specs=[
            pl.BlockSpec((1, HQ, P), lambda b, g: (b, 0, 0)),
            pl.BlockSpec((1, P, HQ), lambda b, g: (b, 0, 0)),
            pl.BlockSpec((1, 1, HQ), lambda b, g: (b, 0, 0)),
            pl.BlockSpec((1, HQ), lambda b, g: (0, 0)),
            pl.BlockSpec((1, GT, 1), lambda b, g: (b, g, 0)),
            pl.BlockSpec((1, GT, P), lambda b, g: (b, g, 0)),
            pl.BlockSpec((HQ, EMB), lambda b, g: (0, 0)),
            pl.BlockSpec((EMB, E), lambda b, g: (0, 0)),
        ],
        out_specs=[
            pl.BlockSpec((GT, EMB), lambda b, g: (b * NG + g, 0)),
            pl.BlockSpec((GT, E), lambda b, g: (b * NG + g, 0)),
            pl.BlockSpec((1, E), lambda b, g: (0, 0)),
        ],
        out_shape=[
            jax.ShapeDtypeStruct((B * G, EMB), F32),
            jax.ShapeDtypeStruct((B * G, E), F32),
            jax.ShapeDtypeStruct((1, E), F32),
        ],
        scratch_shapes=[pltpu.VMEM((GT, HQ), F32)],
    )(kT, v, bq, wl, cap_col, mask, Wcomb, w_gate)


def _moe_body(x_ref, gates_ref, ew1_ref, eb1_ref, ew2_ref, eb2_ref,
              added_ref, ssum_ref, acc_ref):
    n = pl.program_id(0)
    io = jax.lax.broadcasted_iota(jnp.int32, (NT, E), 1)
    gates = gates_ref[...]
    x = x_ref[...]
    for e in range(E):
        gcol = jnp.sum(jnp.where(io == e, gates, 0.0), axis=1, keepdims=True)
        h = jnp.maximum(_mm(x, ew1_ref[e]) + eb1_ref[e], 0.0)
        y = _mm(h, ew2_ref[e]) + eb2_ref[e]
        if e == 0:
            acc_ref[...] = gcol * y
        else:
            acc_ref[...] += gcol * y
    added = x + acc_ref[...]
    added_ref[...] = added
    ls = jnp.sum(added, axis=0, keepdims=True)

    @pl.when(n % TPB == 0)
    def _():
        ssum_ref[0] = ls

    @pl.when(n % TPB > 0)
    def _():
        ssum_ref[0] += ls


def _moe(x, gates, ew1, eb1, ew2, eb2):
    return pl.pallas_call(
        _moe_body,
        grid=(NTILES,),
        in_specs=[
            pl.BlockSpec((NT, EMB), lambda n: (n, 0)),
            pl.BlockSpec((NT, E), lambda n: (n, 0)),
            pl.BlockSpec((E, EMB, HID), lambda n: (0, 0, 0)),
            pl.BlockSpec((E, 1, HID), lambda n: (0, 0, 0)),
            pl.BlockSpec((E, HID, EMB), lambda n: (0, 0, 0)),
            pl.BlockSpec((E, 1, EMB), lambda n: (0, 0, 0)),
        ],
        out_specs=[
            pl.BlockSpec((NT, EMB), lambda n: (n, 0)),
            pl.BlockSpec((1, 1, EMB), lambda n: (n // TPB, 0, 0)),
        ],
        out_shape=[
            jax.ShapeDtypeStruct((B * G, EMB), F32),
            jax.ShapeDtypeStruct((B, 1, EMB), F32),
        ],
        scratch_shapes=[pltpu.VMEM((NT, EMB), F32)],
    )(x, gates, ew1, eb1[:, None, :], ew2, eb2[:, None, :])


def _stats_body(added_ref, ssum_ref, ssq_ref):
    g = pl.program_id(1)
    mu = ssum_ref[0] * (1.0 / G)
    d = added_ref[...] - mu
    lq = jnp.sum(d * d, axis=0, keepdims=True)

    @pl.when(g == 0)
    def _():
        ssq_ref[0] = lq

    @pl.when(g > 0)
    def _():
        ssq_ref[0] += lq


def _stats(added, ssum):
    return pl.pallas_call(
        _stats_body,
        grid=(B, NG),
        in_specs=[
            pl.BlockSpec((GT, EMB), lambda b, g: (b * NG + g, 0)),
            pl.BlockSpec((1, 1, EMB), lambda b, g: (b, 0, 0)),
        ],
        out_specs=pl.BlockSpec((1, 1, EMB), lambda b, g: (b, 0, 0)),
        out_shape=jax.ShapeDtypeStruct((B, 1, EMB), F32),
    )(added, ssum)


def _final_body(added_ref, ssum_ref, ssq_ref, gamma_ref, beta_ref, encT_ref,
                mask_ref, imp_ref, probs_ref, loss_ref):
    b = pl.program_id(0)
    g = pl.program_id(1)
    mu = ssum_ref[0] * (1.0 / G)
    var = ssq_ref[0] * (1.0 / G)
    rstd = jax.lax.rsqrt(var + 1e-5)
    mho = (added_ref[...] - mu) * (rstd * gamma_ref[...]) + beta_ref[...]
    sc = _mm(mho, encT_ref[0]) * (1.0 / SQRT_EMB)
    scm = CLIP * jnp.tanh(sc) + mask_ref[0]
    m = jnp.max(scm, axis=1, keepdims=True)
    ex = jnp.exp(scm - m)
    s = jnp.sum(ex, axis=1, keepdims=True)
    probs_ref[0] = ex / s

    @pl.when(jnp.logical_and(b == 0, g == 0))
    def _():
        im = imp_ref[...]
        mn = jnp.sum(im, axis=1, keepdims=True) * (1.0 / E)
        vr = jnp.sum((im - mn) ** 2, axis=1, keepdims=True) * (1.0 / E)
        loss_ref[...] = vr / (mn * mn + 1e-10)


def _final(added, ssum, ssq, gamma2, beta2, encT, mask, imp):
    return pl.pallas_call(
        _final_body,
        grid=(B, NG),
        in_specs=[
            pl.BlockSpec((GT, EMB), lambda b, g: (b * NG + g, 0)),
            pl.BlockSpec((1, 1, EMB), lambda b, g: (b, 0, 0)),
            pl.BlockSpec((1, 1, EMB), lambda b, g: (b, 0, 0)),
            pl.BlockSpec((1, EMB), lambda b, g: (0, 0)),
            pl.BlockSpec((1, EMB), lambda b, g: (0, 0)),
            pl.BlockSpec((1, EMB, P), lambda b, g: (b, 0, 0)),
            pl.BlockSpec((1, GT, P), lambda b, g: (b, g, 0)),
            pl.BlockSpec((1, E), lambda b, g: (0, 0)),
        ],
        out_specs=[
            pl.BlockSpec((1, GT, P), lambda b, g: (b, g, 0)),
            pl.BlockSpec((1, 1), lambda b, g: (0, 0)),
        ],
        out_shape=[
            jax.ShapeDtypeStruct((B, G, P), F32),
            jax.ShapeDtypeStruct((1, 1), F32),
        ],
    )(added, ssum, ssq, gamma2, beta2, encT, mask, imp)


def kernel(graph, capacity, ninf_mask, encoded_nodes, Wq, Wk, Wv, Wcomb,
           w_gate, ew1, eb1, ew2, eb2, gamma, beta):
    encT = encoded_nodes.transpose(0, 2, 1)
    kT, v, bq = _prep(graph, encoded_nodes, encT, Wk, Wq, Wv)
    wl = Wq[EMB:EMB + 1, :]
    cap_col = capacity[:, :, None]
    mh, gates, imp = _attn(kT, v, bq, wl, cap_col, ninf_mask, Wcomb, w_gate)
    added, ssum = _moe(mh, gates, ew1, eb1, ew2, eb2)
    ssq = _stats(added, ssum)
    probs, loss = _final(added, ssum, ssq, gamma[None, :], beta[None, :],
                         encT, ninf_mask, imp)
    return probs, loss[0, 0]
